# Initial kernel scaffold; baseline (speedup 1.0000x reference)
import jax
import jax.numpy as jnp
from jax.experimental import pallas as pl


def kernel(feats, edge_index, edge_attr, W_rel, b_rel, W_edge, b_edge, W_res, b_res, gamma, beta):
    raise NotImplementedError("write your pallas kernel here")


# speedup vs baseline: 2.7108x; 2.7108x over previous
import functools

import jax
import jax.numpy as jnp
from jax import lax
from jax.experimental import pallas as pl
from jax.experimental.pallas import tpu as pltpu
from jax.experimental.pallas import tpu_sc as plsc

N_NODES = 10000
N_EDGES = 320000
D_IN = 128
D_OUT = 128
D_EDGE = 16

NC = 2
NS = 16
NW = NC * NS
E_PER_W = N_EDGES // NW
CHUNK = 80
N_CHUNKS = E_PER_W // CHUNK
N_PAD = 10240
ROWS_PER_S = N_PAD // NS


def _sc_aggregate(x, src, dst, r, z128):
    mesh = plsc.VectorSubcoreMesh(core_axis_name="c", subcore_axis_name="s")

    @functools.partial(
        pl.kernel,
        out_type=jax.ShapeDtypeStruct((NC, N_PAD, D_OUT), jnp.float32),
        mesh=mesh,
        scratch_types=[
            pltpu.VMEM((CHUNK,), jnp.int32),
            pltpu.VMEM((CHUNK,), jnp.int32),
            pltpu.VMEM((CHUNK, D_OUT), jnp.float32),
            pltpu.VMEM((CHUNK, D_OUT), jnp.float32),
            pltpu.VMEM_SHARED((N_PAD, D_OUT), jnp.float32),
            pltpu.SemaphoreType.DMA,
        ],
    )
    def k(x_hbm, src_hbm, dst_hbm, r_hbm, z128_hbm, acc_out,
          src_v, dst_v, rows_v, r_v, acc_sh, sem):
        c = lax.axis_index("c")
        s = lax.axis_index("s")
        w = c * NS + s

        pltpu.sync_copy(z128_hbm, acc_sh.at[pl.ds(s * ROWS_PER_S, ROWS_PER_S)])
        plsc.subcore_barrier()

        def body(j, carry):
            base = w * E_PER_W + j * CHUNK
            pltpu.sync_copy(src_hbm.at[pl.ds(base, CHUNK)], src_v)
            pltpu.sync_copy(dst_hbm.at[pl.ds(base, CHUNK)], dst_v)
            pltpu.async_copy(x_hbm.at[src_v], rows_v, sem).wait()
            pltpu.sync_copy(r_hbm.at[pl.ds(base, CHUNK)], r_v)
            pltpu.sync_copy(rows_v, acc_sh.at[dst_v], add=True)
            pltpu.sync_copy(r_v, acc_sh.at[dst_v], add=True)
            return carry

        lax.fori_loop(0, N_CHUNKS, body, 0)
        plsc.subcore_barrier()

        pltpu.sync_copy(acc_sh.at[pl.ds(s * ROWS_PER_S, ROWS_PER_S)],
                        acc_out.at[c, pl.ds(s * ROWS_PER_S, ROWS_PER_S)])

    return k(x, src, dst, r, z128)


def _pre_x_body(feats_ref, wr_ref, br_ref, x_ref):
    x_ref[...] = jnp.dot(feats_ref[...], wr_ref[...],
                         preferred_element_type=jnp.float32) + br_ref[...]


def _pre_r_body(ea_ref, we_ref, be_ref, r_ref):
    r_ref[...] = jnp.dot(ea_ref[...], we_ref[...],
                         preferred_element_type=jnp.float32) + be_ref[...]


def _post_body(acc_ref, feats_ref, wres_ref, bres_ref, gamma_ref, beta_ref,
               out_ref):
    agg = (acc_ref[0] + acc_ref[1])[:N_NODES]
    res = jnp.dot(feats_ref[...], wres_ref[...],
                  preferred_element_type=jnp.float32) + bres_ref[...]
    y = jnp.maximum(agg, 0.0) + jnp.maximum(res, 0.0)
    mean = jnp.mean(y, axis=0, keepdims=True)
    var = jnp.mean((y - mean) ** 2, axis=0, keepdims=True)
    out_ref[...] = ((y - mean) * lax.rsqrt(var + 1e-5) * gamma_ref[...]
                    + beta_ref[...])


R_BLK = 8000


def kernel(feats, edge_index, edge_attr, W_rel, b_rel, W_edge, b_edge,
           W_res, b_res, gamma, beta):
    ei = edge_index.astype(jnp.int32)
    src = ei[0]
    dst = ei[1]
    z128 = jnp.zeros((ROWS_PER_S, D_OUT), jnp.float32)

    x = pl.pallas_call(
        _pre_x_body,
        out_shape=jax.ShapeDtypeStruct((N_NODES, D_OUT), jnp.float32),
    )(feats, W_rel, b_rel.reshape(1, D_OUT))

    r = pl.pallas_call(
        _pre_r_body,
        grid=(N_EDGES // R_BLK,),
        in_specs=[
            pl.BlockSpec((R_BLK, D_EDGE), lambda i: (i, 0)),
            pl.BlockSpec((D_EDGE, D_OUT), lambda i: (0, 0)),
            pl.BlockSpec((1, D_OUT), lambda i: (0, 0)),
        ],
        out_specs=pl.BlockSpec((R_BLK, D_OUT), lambda i: (i, 0)),
        out_shape=jax.ShapeDtypeStruct((N_EDGES, D_OUT), jnp.float32),
    )(edge_attr, W_edge, b_edge.reshape(1, D_OUT))

    acc = _sc_aggregate(x, src, dst, r, z128)

    out = pl.pallas_call(
        _post_body,
        out_shape=jax.ShapeDtypeStruct((N_NODES, D_OUT), jnp.float32),
    )(acc, feats, W_res, b_res.reshape(1, D_OUT),
      gamma.reshape(1, D_OUT), beta.reshape(1, D_OUT))
    return out
